# baseline (speedup 1.0000x reference)
import jax
import jax.numpy as jnp
from jax.experimental import pallas as pl
from jax.experimental.pallas import tpu as pltpu

N = 4096
D = 64
BM = 512
BK = 1024


def _matmul_block(inp_ref, w_ref, out_ref):
    k = pl.program_id(1)
    part = jnp.dot(inp_ref[...], w_ref[...],
                   preferred_element_type=jnp.float32)

    @pl.when(k == 0)
    def _():
        out_ref[...] = part

    @pl.when(k != 0)
    def _():
        out_ref[...] += part


@jax.jit
def kernel(inp, weight):
    grid = (N // BM, N // BK)
    return pl.pallas_call(
        _matmul_block,
        grid=grid,
        in_specs=[
            pl.BlockSpec((BM, BK), lambda i, k: (i, k)),
            pl.BlockSpec((BK, D), lambda i, k: (k, 0)),
        ],
        out_specs=pl.BlockSpec((BM, D), lambda i, k: (i, 0)),
        out_shape=jax.ShapeDtypeStruct((N, D), jnp.float32),
        compiler_params=pltpu.CompilerParams(
            dimension_semantics=("arbitrary", "arbitrary"),
        ),
    )(inp, weight)


# speedup vs baseline: 1.4232x; 1.4232x over previous
import jax
import jax.numpy as jnp
from jax.experimental import pallas as pl
from jax.experimental.pallas import tpu as pltpu

N = 4096
D = 64
BM = 256
NBLK = N // BM
NBUF = 12


def _spmm_kernel(inp_hbm, w_ref, out_ref, bufs, sems):
    def copy(i):
        return pltpu.make_async_copy(
            inp_hbm.at[pl.ds(i * BM, BM), :], bufs.at[i % NBUF],
            sems.at[i % NBUF])

    for i in range(NBUF):
        copy(i).start()
    for i in range(NBLK):
        copy(i).wait()
        out_ref[pl.ds(i * BM, BM), :] = jnp.dot(
            bufs[i % NBUF], w_ref[...], preferred_element_type=jnp.float32)
        if i + NBUF < NBLK:
            copy(i + NBUF).start()


@jax.jit
def kernel(inp, weight):
    return pl.pallas_call(
        _spmm_kernel,
        in_specs=[
            pl.BlockSpec(memory_space=pltpu.MemorySpace.HBM),
            pl.BlockSpec(memory_space=pltpu.MemorySpace.VMEM),
        ],
        out_specs=pl.BlockSpec(memory_space=pltpu.MemorySpace.VMEM),
        out_shape=jax.ShapeDtypeStruct((N, D), jnp.float32),
        scratch_shapes=[
            pltpu.VMEM((NBUF, BM, N), jnp.float32),
            pltpu.SemaphoreType.DMA((NBUF,)),
        ],
    )(inp, weight)
